# baseline (speedup 1.0000x reference)
import functools

import jax
import jax.numpy as jnp
from jax import lax
from jax.experimental import pallas as pl
from jax.experimental.pallas import tpu as pltpu
from jax.experimental.pallas import tpu_sc as plsc

_NUM_WORKERS = 32
_RING = 4
_LEAD = 2


def kernel(inputs, token_table, pos_table):
    B, S = inputs.shape
    V, D = token_table.shape
    seq_per_w = B // _NUM_WORKERS
    n_units = seq_per_w * 2
    sizes = (96, S - 96)
    offs = (0, 96)
    bufsz = max(sizes)
    n_loop = (n_units - _RING // 2) // _RING

    idx1 = inputs.astype(jnp.int32).reshape(-1)

    mesh = plsc.VectorSubcoreMesh(core_axis_name="c", subcore_axis_name="s")

    @functools.partial(
        pl.kernel,
        mesh=mesh,
        out_type=jax.ShapeDtypeStruct((B, S, D), jnp.float32),
        scratch_types=(
            [pltpu.VMEM((seq_per_w * S,), jnp.int32),
             pltpu.VMEM((S, D), jnp.float32)]
            + [pltpu.VMEM((bufsz, D), jnp.float32)] * _RING
            + [pltpu.SemaphoreType.DMA] * (2 * _RING)
        ),
    )
    def emb_kernel(idx_hbm, tok_hbm, pos_hbm, out_hbm, idx_v, pos_v, *rest):
        bufs = rest[:_RING]
        gsems = rest[_RING:2 * _RING]
        ssems = rest[2 * _RING:]
        wid = lax.axis_index("s") * 2 + lax.axis_index("c")
        seq0 = wid * seq_per_w
        pltpu.sync_copy(idx_hbm.at[pl.ds(seq0 * S, seq_per_w * S)], idx_v)
        pltpu.sync_copy(pos_hbm, pos_v)

        def issue_gather(u, p, r):
            pltpu.async_copy(
                tok_hbm.at[idx_v.at[pl.ds((u // 2) * S + offs[p], sizes[p])]],
                bufs[r].at[pl.ds(0, sizes[p])], gsems[r])

        def wait_gather(p, r):
            pltpu.make_async_copy(
                tok_hbm.at[idx_v.at[pl.ds(offs[p], sizes[p])]],
                bufs[r].at[pl.ds(0, sizes[p])], gsems[r]).wait()

        def issue_store(u, p, r):
            pltpu.async_copy(
                bufs[r].at[pl.ds(0, sizes[p])],
                out_hbm.at[seq0 + u // 2, pl.ds(offs[p], sizes[p])],
                ssems[r])

        def drain_store(p, r):
            pltpu.make_async_copy(
                bufs[r].at[pl.ds(0, sizes[p])],
                out_hbm.at[0, pl.ds(offs[p], sizes[p])], ssems[r]).wait()

        def add_pos(p, r):
            buf = bufs[r]

            def body(i, c):
                for k in range(4):
                    rr = i * 4 + k
                    for c8 in range(D // 16):
                        sl = pl.ds(c8 * 16, 16)
                        plsc.addupdate(buf.at[rr, sl],
                                       pos_v[offs[p] + rr, sl])
                return c
            lax.fori_loop(0, sizes[p] // 4, body, 0)

        for u0 in range(_LEAD):
            issue_gather(u0, u0 % 2, u0)

        def step(t, carry):
            for k in range(_RING):
                u = t * _RING + k
                p = k % 2
                rg = (k + _LEAD) % _RING
                pg = (k + _LEAD) % 2
                if k < _LEAD:
                    pl.when(t >= 1)(lambda p2=pg, r2=rg: drain_store(p2, r2))
                else:
                    drain_store(pg, rg)
                issue_gather(u + _LEAD, pg, rg)
                wait_gather(p, k)
                add_pos(p, k)
                issue_store(u, p, k)
            return carry

        lax.fori_loop(0, n_loop, step, 0)

        for u in range(n_loop * _RING, n_units):
            r = u % _RING
            p = u % 2
            if u + _LEAD < n_units:
                rg = (u + _LEAD) % _RING
                pg = (u + _LEAD) % 2
                drain_store(pg, rg)
                issue_gather(u + _LEAD, pg, rg)
            wait_gather(p, r)
            add_pos(p, r)
            issue_store(u, p, r)
        for u in range(n_units - _RING, n_units):
            drain_store(u % 2, u % _RING)

    return emb_kernel(idx1, token_table, pos_table)
